# baseline (speedup 1.0000x reference)
import jax
import jax.numpy as jnp
from jax.experimental import pallas as pl
from jax.experimental.pallas import tpu as pltpu

_BBLK = 64


def _ceil128(n):
    return -(-n // 128) * 128


def _band(w, s_in):
    cin, cout = w.shape[2], w.shape[3]
    wpad = s_in + 4
    sp = (s_in + 2) // 2
    wpc = sp * cout
    hh = _ceil128(wpc)
    wf = w.astype(jnp.float32)
    wv = jnp.arange(wpad)[:, None, None, None]
    tgt = (2 * jnp.arange(sp)[None, None, :, None]
           + jnp.arange(2)[None, :, None, None]
           + jnp.arange(3)[None, None, None, :])
    sel = (wv == tgt).astype(jnp.float32)
    band = jnp.einsum('wqpk,hkio->hwiqpo', sel, wf)
    band = band.reshape(3, wpad * cin, 2, wpc)
    band = jnp.pad(band, ((0, 0), (0, 0), (0, 0), (0, hh - wpc)))
    kp = _ceil128(wpad * cin)
    band = jnp.pad(band, ((0, 0), (0, kp - wpad * cin), (0, 0), (0, 0)))
    return band.reshape(3 * kp, 2 * hh).astype(jnp.bfloat16)


def _band_bias(b, s_in):
    cout = b.shape[0]
    sp = (s_in + 2) // 2
    wpc = sp * cout
    hh = _ceil128(wpc)
    bb = jnp.pad(jnp.tile(b.astype(jnp.float32), sp), (0, hh - wpc))
    return jnp.tile(bb, 2)


def _banded_matmul_pool(t, band, bias, bblk, cout, sp):
    hh = band.shape[1] // 2
    wpc = sp * cout
    acc = jnp.dot(t, band, preferred_element_type=jnp.float32)
    acc = acc + bias
    y = jnp.maximum(acc[:, :hh], acc[:, hh:])[:, :wpc]
    y = y.reshape(bblk, 2 * sp, wpc)
    y = y.reshape(bblk, sp, 2, wpc).max(axis=2)
    return y.astype(jnp.bfloat16)


def _kh_pieces(p, c, s_out):
    bblk, s_in, _ = p.shape
    wpad_c = (s_in + 4) * c
    kp = _ceil128(wpad_c)
    h = 2 * s_out
    pieces = []
    for s in range(3):
        lo, hi = s - 2, s - 2 + h
        top = max(0, -lo)
        seg = p[:, max(0, lo):min(s_in, hi)]
        bot = h - top - (min(s_in, hi) - max(0, lo))
        q = jnp.pad(seg, ((0, 0), (top, bot), (2 * c, kp - wpad_c + 2 * c)))
        pieces.append(q.reshape(bblk * h, kp))
    return jnp.concatenate(pieces, axis=1)


def _direct_layer(p4, w, bias, cout, sp):
    bblk, s_in, _, cin = p4.shape
    h = 2 * sp
    r = bblk * h * h
    t = jnp.pad(p4, ((0, 0), (2, 2), (2, 2), (0, 0)))
    pieces = [t[:, kh:kh + h, kw:kw + h, :].reshape(r, cin)
              for kh in range(3) for kw in range(3)]
    tt = jnp.concatenate(pieces, axis=1)
    acc = jnp.dot(tt, w, preferred_element_type=jnp.float32)[:, :cout]
    acc = acc + bias
    y = acc.reshape(bblk, h, h, cout)
    y = y.reshape(bblk, sp, 2, h, cout).max(axis=2)
    y = y.reshape(bblk, sp, sp, 2, cout).max(axis=3)
    return y.astype(jnp.bfloat16)


def _fused_kernel(x_ref, bd_ref, wx_ref, bs_ref, o_ref):
    bblk = x_ref.shape[0]
    t0 = x_ref[...].reshape(bblk * 34, 384)
    p0 = _banded_matmul_pool(t0, bd_ref[0:384], bs_ref[0:1], bblk, 32, 17)
    t1 = _kh_pieces(p0, 32, 9)
    p1 = _banded_matmul_pool(t1, bd_ref[384:2688], bs_ref[1:2], bblk, 64, 9)
    t2 = _kh_pieces(p1, 64, 5)
    p2 = _banded_matmul_pool(t2, bd_ref[2688:5376], bs_ref[2:3], bblk, 128, 5)
    p2w = jnp.stack([p2[:, :, w * 128:(w + 1) * 128] for w in range(5)], axis=2)
    p3 = _direct_layer(p2w, wx_ref[0:1152], bs_ref[3:4, :256], 256, 3)
    p4 = _direct_layer(p3, wx_ref[1152:3456], bs_ref[4:5, :512], 512, 2)
    ha = None
    hb = None
    for idx, (hh, ww) in enumerate(((0, 0), (0, 1), (1, 0), (1, 1))):
        xp = p4[:, hh, ww, :]
        da = jnp.dot(xp, wx_ref[3456 + 512 * idx:3456 + 512 * (idx + 1)],
                     preferred_element_type=jnp.float32)
        db = jnp.dot(xp, wx_ref[5504 + 512 * idx:5504 + 512 * (idx + 1)],
                     preferred_element_type=jnp.float32)
        ha = da if ha is None else ha + da
        hb = db if hb is None else hb + db
    hcat = jnp.concatenate([ha, hb], axis=1)
    hcat = (hcat + bs_ref[5:6, :1024]).astype(jnp.bfloat16)
    out = jnp.dot(hcat, wx_ref[7552:8576, :128],
                  preferred_element_type=jnp.float32)
    o_ref[...] = out + bs_ref[6:7, :128]


def kernel(x, conv0_w, conv0_b, conv1_w, conv1_b, conv2_w, conv2_b,
           conv3_w, conv3_b, conv4_w, conv4_b, fc1_w, fc1_b, fc2_w, fc2_b):
    b = x.shape[0]
    bblk = _BBLK
    xh = jnp.transpose(x, (0, 2, 3, 1)).astype(jnp.bfloat16)
    xh = jnp.pad(xh, ((0, 0), (2, 2), (2, 2), (0, 0)))
    xh = xh.reshape(b, 36, 108)
    t0 = jnp.concatenate(
        [jnp.pad(xh[:, s:s + 34], ((0, 0), (0, 0), (0, 20))) for s in range(3)],
        axis=2)

    bd = jnp.concatenate(
        [_band(conv0_w, 32), _band(conv1_w, 17), _band(conv2_w, 9)], axis=0)
    w3 = jnp.pad(conv3_w.reshape(9 * 128, 256), ((0, 0), (0, 256)))
    w4 = conv4_w.reshape(9 * 256, 512)
    fw1 = fc1_w.reshape(512, 2, 2, 1024).transpose(1, 2, 0, 3).reshape(2048, 1024)
    fw2 = jnp.pad(fc2_w, ((0, 0), (0, 502)))
    wx = jnp.concatenate([w3, w4, fw1[:, :512], fw1[:, 512:], fw2], axis=0)
    bs = jnp.stack([
        _band_bias(conv0_b, 32),
        _band_bias(conv1_b, 17),
        _band_bias(conv2_b, 9),
        jnp.pad(conv3_b.astype(jnp.float32), (0, 1024)),
        jnp.pad(conv4_b.astype(jnp.float32), (0, 768)),
        jnp.pad(fc1_b.astype(jnp.float32), (0, 256)),
        jnp.pad(fc2_b.astype(jnp.float32), (0, 1270)),
    ], axis=0)

    vmem = pl.BlockSpec(memory_space=pltpu.VMEM)
    out = pl.pallas_call(
        _fused_kernel,
        out_shape=jax.ShapeDtypeStruct((b, 128), jnp.float32),
        grid=(b // bblk,),
        in_specs=[pl.BlockSpec((bblk, 34, 384), lambda i: (i, 0, 0))] + [vmem] * 3,
        out_specs=pl.BlockSpec((bblk, 128), lambda i: (i, 0)),
        compiler_params=pltpu.CompilerParams(
            dimension_semantics=("parallel",),
        ),
        name="fused_cnn",
    )(t0, bd, wx, bs)
    return out[:, :10]


# speedup vs baseline: 1.0478x; 1.0478x over previous
import jax
import jax.numpy as jnp
from jax.experimental import pallas as pl
from jax.experimental.pallas import tpu as pltpu

_BBLK = 16


def _ceil128(n):
    return -(-n // 128) * 128


def _band(w, s_in):
    cin, cout = w.shape[2], w.shape[3]
    wpad = s_in + 4
    sp = (s_in + 2) // 2
    wpc = sp * cout
    hh = _ceil128(wpc)
    wf = w.astype(jnp.float32)
    wv = jnp.arange(wpad)[:, None, None, None]
    tgt = (2 * jnp.arange(sp)[None, None, :, None]
           + jnp.arange(2)[None, :, None, None]
           + jnp.arange(3)[None, None, None, :])
    sel = (wv == tgt).astype(jnp.float32)
    band = jnp.einsum('wqpk,hkio->hwiqpo', sel, wf)
    band = band.reshape(3, wpad * cin, 2, wpc)
    band = jnp.pad(band, ((0, 0), (0, 0), (0, 0), (0, hh - wpc)))
    kp = _ceil128(wpad * cin)
    band = jnp.pad(band, ((0, 0), (0, kp - wpad * cin), (0, 0), (0, 0)))
    return band.reshape(3 * kp, 2 * hh).astype(jnp.bfloat16)


def _band_bias(b, s_in):
    cout = b.shape[0]
    sp = (s_in + 2) // 2
    wpc = sp * cout
    hh = _ceil128(wpc)
    bb = jnp.pad(jnp.tile(b.astype(jnp.float32), sp), (0, hh - wpc))
    return jnp.tile(bb, 2)


def _banded_matmul_pool(t, band, bias, bblk, cout, sp):
    hh = band.shape[1] // 2
    wpc = sp * cout
    acc = jnp.dot(t, band, preferred_element_type=jnp.float32)
    acc = acc + bias
    y = jnp.maximum(acc[:, :hh], acc[:, hh:])[:, :wpc]
    y = y.reshape(bblk, 2 * sp, wpc)
    y = y.reshape(bblk, sp, 2, wpc).max(axis=2)
    return y.astype(jnp.bfloat16)


def _kh_pieces(p, c, s_out):
    bblk, s_in, _ = p.shape
    wpad_c = (s_in + 4) * c
    kp = _ceil128(wpad_c)
    h = 2 * s_out
    pieces = []
    for s in range(3):
        lo, hi = s - 2, s - 2 + h
        top = max(0, -lo)
        seg = p[:, max(0, lo):min(s_in, hi)]
        bot = h - top - (min(s_in, hi) - max(0, lo))
        q = jnp.pad(seg, ((0, 0), (top, bot), (2 * c, kp - wpad_c + 2 * c)))
        pieces.append(q.reshape(bblk * h, kp))
    return jnp.concatenate(pieces, axis=1)


def _direct_layer(p4, w, bias, cout, sp):
    bblk, s_in, _, cin = p4.shape
    h = 2 * sp
    r = bblk * h * h
    t = jnp.pad(p4, ((0, 0), (2, 2), (2, 2), (0, 0)))
    pieces = [t[:, kh:kh + h, kw:kw + h, :].reshape(r, cin)
              for kh in range(3) for kw in range(3)]
    tt = jnp.concatenate(pieces, axis=1)
    acc = jnp.dot(tt, w, preferred_element_type=jnp.float32)[:, :cout]
    acc = acc + bias
    y = acc.reshape(bblk, h, h, cout)
    y = y.reshape(bblk, sp, 2, h, cout).max(axis=2)
    y = y.reshape(bblk, sp, sp, 2, cout).max(axis=3)
    return y.astype(jnp.bfloat16)


def _fused_kernel(x_ref, bd_ref, wx_ref, bs_ref, o_ref):
    bblk = x_ref.shape[0]
    t0 = x_ref[...].reshape(bblk * 34, 384)
    p0 = _banded_matmul_pool(t0, bd_ref[0:384], bs_ref[0:1], bblk, 32, 17)
    t1 = _kh_pieces(p0, 32, 9)
    p1 = _banded_matmul_pool(t1, bd_ref[384:2688], bs_ref[1:2], bblk, 64, 9)
    t2 = _kh_pieces(p1, 64, 5)
    p2 = _banded_matmul_pool(t2, bd_ref[2688:5376], bs_ref[2:3], bblk, 128, 5)
    p2w = jnp.stack([p2[:, :, w * 128:(w + 1) * 128] for w in range(5)], axis=2)
    p3 = _direct_layer(p2w, wx_ref[0:1152], bs_ref[3:4, :256], 256, 3)
    p4 = _direct_layer(p3, wx_ref[1152:3456], bs_ref[4:5, :512], 512, 2)
    ha = None
    hb = None
    for idx, (hh, ww) in enumerate(((0, 0), (0, 1), (1, 0), (1, 1))):
        xp = p4[:, hh, ww, :]
        da = jnp.dot(xp, wx_ref[3456 + 512 * idx:3456 + 512 * (idx + 1)],
                     preferred_element_type=jnp.float32)
        db = jnp.dot(xp, wx_ref[5504 + 512 * idx:5504 + 512 * (idx + 1)],
                     preferred_element_type=jnp.float32)
        ha = da if ha is None else ha + da
        hb = db if hb is None else hb + db
    hcat = jnp.concatenate([ha, hb], axis=1)
    hcat = (hcat + bs_ref[5:6, :1024]).astype(jnp.bfloat16)
    out = jnp.dot(hcat, wx_ref[7552:8576, :128],
                  preferred_element_type=jnp.float32)
    o_ref[...] = out + bs_ref[6:7, :128]


def kernel(x, conv0_w, conv0_b, conv1_w, conv1_b, conv2_w, conv2_b,
           conv3_w, conv3_b, conv4_w, conv4_b, fc1_w, fc1_b, fc2_w, fc2_b):
    b = x.shape[0]
    bblk = _BBLK
    xh = jnp.transpose(x, (0, 2, 3, 1)).astype(jnp.bfloat16)
    xh = jnp.pad(xh, ((0, 0), (2, 2), (2, 2), (0, 0)))
    xh = xh.reshape(b, 36, 108)
    t0 = jnp.concatenate(
        [jnp.pad(xh[:, s:s + 34], ((0, 0), (0, 0), (0, 20))) for s in range(3)],
        axis=2)

    bd = jnp.concatenate(
        [_band(conv0_w, 32), _band(conv1_w, 17), _band(conv2_w, 9)], axis=0)
    w3 = jnp.pad(conv3_w.reshape(9 * 128, 256), ((0, 0), (0, 256)))
    w4 = conv4_w.reshape(9 * 256, 512)
    fw1 = fc1_w.reshape(512, 2, 2, 1024).transpose(1, 2, 0, 3).reshape(2048, 1024)
    fw2 = jnp.pad(fc2_w, ((0, 0), (0, 502)))
    wx = jnp.concatenate([w3, w4, fw1[:, :512], fw1[:, 512:], fw2], axis=0)
    bs = jnp.stack([
        _band_bias(conv0_b, 32),
        _band_bias(conv1_b, 17),
        _band_bias(conv2_b, 9),
        jnp.pad(conv3_b.astype(jnp.float32), (0, 1024)),
        jnp.pad(conv4_b.astype(jnp.float32), (0, 768)),
        jnp.pad(fc1_b.astype(jnp.float32), (0, 256)),
        jnp.pad(fc2_b.astype(jnp.float32), (0, 1270)),
    ], axis=0)

    vmem = pl.BlockSpec(memory_space=pltpu.VMEM)
    out = pl.pallas_call(
        _fused_kernel,
        out_shape=jax.ShapeDtypeStruct((b, 128), jnp.float32),
        grid=(b // bblk,),
        in_specs=[pl.BlockSpec((bblk, 34, 384), lambda i: (i, 0, 0))] + [vmem] * 3,
        out_specs=pl.BlockSpec((bblk, 128), lambda i: (i, 0)),
        compiler_params=pltpu.CompilerParams(
            dimension_semantics=("parallel",),
        ),
        name="fused_cnn",
    )(t0, bd, wx, bs)
    return out[:, :10]


# speedup vs baseline: 1.1304x; 1.0788x over previous
import jax
import jax.numpy as jnp
from jax.experimental import pallas as pl
from jax.experimental.pallas import tpu as pltpu

_BBLK = 32


def _ceil128(n):
    return -(-n // 128) * 128


def _band(w, s_in):
    cin, cout = w.shape[2], w.shape[3]
    wpad = s_in + 4
    sp = (s_in + 2) // 2
    wpc = sp * cout
    hh = _ceil128(wpc)
    wf = w.astype(jnp.float32)
    wv = jnp.arange(wpad)[:, None, None, None]
    tgt = (2 * jnp.arange(sp)[None, None, :, None]
           + jnp.arange(2)[None, :, None, None]
           + jnp.arange(3)[None, None, None, :])
    sel = (wv == tgt).astype(jnp.float32)
    band = jnp.einsum('wqpk,hkio->hwiqpo', sel, wf)
    band = band.reshape(3, wpad * cin, 2, wpc)
    band = jnp.pad(band, ((0, 0), (0, 0), (0, 0), (0, hh - wpc)))
    kp = _ceil128(wpad * cin)
    band = jnp.pad(band, ((0, 0), (0, kp - wpad * cin), (0, 0), (0, 0)))
    return band.reshape(3 * kp, 2 * hh).astype(jnp.bfloat16)


def _band_bias(b, s_in):
    cout = b.shape[0]
    sp = (s_in + 2) // 2
    wpc = sp * cout
    hh = _ceil128(wpc)
    bb = jnp.pad(jnp.tile(b.astype(jnp.float32), sp), (0, hh - wpc))
    return jnp.tile(bb, 2)


def _banded_matmul_pool(t, band, bias, bblk, cout, sp):
    hh = band.shape[1] // 2
    wpc = sp * cout
    acc = jnp.dot(t, band, preferred_element_type=jnp.float32)
    acc = acc + bias
    y = jnp.maximum(acc[:, :hh], acc[:, hh:])[:, :wpc]
    y = y.reshape(bblk, 2 * sp, wpc)
    y = y.reshape(bblk, sp, 2, wpc).max(axis=2)
    return y.astype(jnp.bfloat16)


def _kh_pieces(p, c, s_out):
    bblk, s_in, _ = p.shape
    wpad_c = (s_in + 4) * c
    kp = _ceil128(wpad_c)
    h = 2 * s_out
    pieces = []
    for s in range(3):
        lo, hi = s - 2, s - 2 + h
        top = max(0, -lo)
        seg = p[:, max(0, lo):min(s_in, hi)]
        bot = h - top - (min(s_in, hi) - max(0, lo))
        q = jnp.pad(seg, ((0, 0), (top, bot), (2 * c, kp - wpad_c + 2 * c)))
        pieces.append(q.reshape(bblk * h, kp))
    return jnp.concatenate(pieces, axis=1)


def _direct_layer(p4, w, bias, cout, sp):
    bblk, s_in, _, cin = p4.shape
    h = 2 * sp
    r = bblk * h * h
    t = jnp.pad(p4, ((0, 0), (2, 2), (2, 2), (0, 0)))
    pieces = [t[:, kh:kh + h, kw:kw + h, :].reshape(r, cin)
              for kh in range(3) for kw in range(3)]
    tt = jnp.concatenate(pieces, axis=1)
    acc = jnp.dot(tt, w, preferred_element_type=jnp.float32)[:, :cout]
    acc = acc + bias
    y = acc.reshape(bblk, h, h, cout)
    y = y.reshape(bblk, sp, 2, h, cout).max(axis=2)
    y = y.reshape(bblk, sp, sp, 2, cout).max(axis=3)
    return y.astype(jnp.bfloat16)


def _fused_kernel(x_ref, bd_ref, wx_ref, bs_ref, o_ref):
    bblk = x_ref.shape[0]
    t0 = x_ref[...].reshape(bblk * 34, 384)
    p0 = _banded_matmul_pool(t0, bd_ref[0:384], bs_ref[0:1], bblk, 32, 17)
    t1 = _kh_pieces(p0, 32, 9)
    p1 = _banded_matmul_pool(t1, bd_ref[384:2688], bs_ref[1:2], bblk, 64, 9)
    t2 = _kh_pieces(p1, 64, 5)
    p2 = _banded_matmul_pool(t2, bd_ref[2688:5376], bs_ref[2:3], bblk, 128, 5)
    p2w = jnp.stack([p2[:, :, w * 128:(w + 1) * 128] for w in range(5)], axis=2)
    p3 = _direct_layer(p2w, wx_ref[0:1152], bs_ref[3:4, :256], 256, 3)
    p4 = _direct_layer(p3, wx_ref[1152:3456], bs_ref[4:5, :512], 512, 2)
    ha = None
    hb = None
    for idx, (hh, ww) in enumerate(((0, 0), (0, 1), (1, 0), (1, 1))):
        xp = p4[:, hh, ww, :]
        da = jnp.dot(xp, wx_ref[3456 + 512 * idx:3456 + 512 * (idx + 1)],
                     preferred_element_type=jnp.float32)
        db = jnp.dot(xp, wx_ref[5504 + 512 * idx:5504 + 512 * (idx + 1)],
                     preferred_element_type=jnp.float32)
        ha = da if ha is None else ha + da
        hb = db if hb is None else hb + db
    hcat = jnp.concatenate([ha, hb], axis=1)
    hcat = (hcat + bs_ref[5:6, :1024]).astype(jnp.bfloat16)
    out = jnp.dot(hcat, wx_ref[7552:8576, :128],
                  preferred_element_type=jnp.float32)
    o_ref[...] = out + bs_ref[6:7, :128]


def kernel(x, conv0_w, conv0_b, conv1_w, conv1_b, conv2_w, conv2_b,
           conv3_w, conv3_b, conv4_w, conv4_b, fc1_w, fc1_b, fc2_w, fc2_b):
    b = x.shape[0]
    bblk = _BBLK
    xh = jnp.transpose(x, (0, 2, 3, 1)).astype(jnp.bfloat16)
    xh = jnp.pad(xh, ((0, 0), (2, 2), (2, 2), (0, 0)))
    xh = xh.reshape(b, 36, 108)
    t0 = jnp.concatenate(
        [jnp.pad(xh[:, s:s + 34], ((0, 0), (0, 0), (0, 20))) for s in range(3)],
        axis=2)

    bd = jnp.concatenate(
        [_band(conv0_w, 32), _band(conv1_w, 17), _band(conv2_w, 9)], axis=0)
    w3 = jnp.pad(conv3_w.reshape(9 * 128, 256), ((0, 0), (0, 256)))
    w4 = conv4_w.reshape(9 * 256, 512)
    fw1 = fc1_w.reshape(512, 2, 2, 1024).transpose(1, 2, 0, 3).reshape(2048, 1024)
    fw2 = jnp.pad(fc2_w, ((0, 0), (0, 502)))
    wx = jnp.concatenate([w3, w4, fw1[:, :512], fw1[:, 512:], fw2], axis=0)
    bs = jnp.stack([
        _band_bias(conv0_b, 32),
        _band_bias(conv1_b, 17),
        _band_bias(conv2_b, 9),
        jnp.pad(conv3_b.astype(jnp.float32), (0, 1024)),
        jnp.pad(conv4_b.astype(jnp.float32), (0, 768)),
        jnp.pad(fc1_b.astype(jnp.float32), (0, 256)),
        jnp.pad(fc2_b.astype(jnp.float32), (0, 1270)),
    ], axis=0)

    vmem = pl.BlockSpec(memory_space=pltpu.VMEM)
    out = pl.pallas_call(
        _fused_kernel,
        out_shape=jax.ShapeDtypeStruct((b, 128), jnp.float32),
        grid=(b // bblk,),
        in_specs=[pl.BlockSpec((bblk, 34, 384), lambda i: (i, 0, 0))] + [vmem] * 3,
        out_specs=pl.BlockSpec((bblk, 128), lambda i: (i, 0)),
        compiler_params=pltpu.CompilerParams(
            dimension_semantics=("parallel",),
        ),
        name="fused_cnn",
    )(t0, bd, wx, bs)
    return out[:, :10]


# speedup vs baseline: 1.2029x; 1.0641x over previous
import jax
import jax.numpy as jnp
from jax.experimental import pallas as pl
from jax.experimental.pallas import tpu as pltpu

_BBLK = 32


def _ceil128(n):
    return -(-n // 128) * 128


def _band(w, s_in):
    cin, cout = w.shape[2], w.shape[3]
    wpad = s_in + 4
    sp = (s_in + 2) // 2
    wpc = sp * cout
    hh = _ceil128(wpc)
    wf = w.astype(jnp.float32)
    wv = jnp.arange(wpad)[:, None, None, None]
    tgt = (2 * jnp.arange(sp)[None, None, :, None]
           + jnp.arange(2)[None, :, None, None]
           + jnp.arange(3)[None, None, None, :])
    sel = (wv == tgt).astype(jnp.float32)
    band = jnp.einsum('wqpk,hkio->hwiqpo', sel, wf)
    band = band.reshape(3, wpad * cin, 2, wpc)
    band = jnp.pad(band, ((0, 0), (0, 0), (0, 0), (0, hh - wpc)))
    kp = _ceil128(wpad * cin)
    band = jnp.pad(band, ((0, 0), (0, kp - wpad * cin), (0, 0), (0, 0)))
    return band.reshape(3 * kp, 2 * hh).astype(jnp.bfloat16)


def _band_bias(b, s_in):
    cout = b.shape[0]
    sp = (s_in + 2) // 2
    wpc = sp * cout
    hh = _ceil128(wpc)
    bb = jnp.pad(jnp.tile(b.astype(jnp.float32), sp), (0, hh - wpc))
    return jnp.tile(bb, 2)


def _banded_matmul_pool(t, band, bias, bblk, cout, sp, rows=None):
    rows = 2 * sp if rows is None else rows
    hh = band.shape[1] // 2
    wpc = sp * cout
    acc = jnp.dot(t, band, preferred_element_type=jnp.float32)
    acc = acc + bias
    y = jnp.maximum(acc[:, :hh], acc[:, hh:])[:, :wpc]
    y = y.reshape(bblk, rows, wpc)[:, :2 * sp]
    y = y.reshape(bblk, sp, 2, wpc).max(axis=2)
    return y.astype(jnp.bfloat16)


def _kh_pieces(p, c, s_out):
    bblk, s_in, _ = p.shape
    wpad_c = (s_in + 4) * c
    kp = _ceil128(wpad_c)
    h = 2 * s_out
    pieces = []
    for s in range(3):
        lo, hi = s - 2, s - 2 + h
        top = max(0, -lo)
        seg = p[:, max(0, lo):min(s_in, hi)]
        bot = h - top - (min(s_in, hi) - max(0, lo))
        q = jnp.pad(seg, ((0, 0), (top, bot), (2 * c, kp - wpad_c + 2 * c)))
        pieces.append(q.reshape(bblk * h, kp))
    return jnp.concatenate(pieces, axis=1)


def _direct_layer(p4, w, bias, cout, sp):
    bblk, s_in, _, cin = p4.shape
    h = 2 * sp
    r = bblk * h * h
    t = jnp.pad(p4, ((0, 0), (2, 2), (2, 2), (0, 0)))
    pieces = [t[:, kh:kh + h, kw:kw + h, :].reshape(r, cin)
              for kh in range(3) for kw in range(3)]
    tt = jnp.concatenate(pieces, axis=1)
    acc = jnp.dot(tt, w, preferred_element_type=jnp.float32)[:, :cout]
    acc = acc + bias
    y = acc.reshape(bblk, h, h, cout)
    y = y.reshape(bblk, sp, 2, h, cout).max(axis=2)
    y = y.reshape(bblk, sp, sp, 2, cout).max(axis=3)
    return y.astype(jnp.bfloat16)


def _fused_kernel(x_ref, bd_ref, wx_ref, bs_ref, o_ref):
    bblk = x_ref.shape[0]
    r0 = bblk * 36
    xf = jnp.pad(x_ref[...].reshape(r0, 108), ((0, 2), (0, 20)))
    t0 = jnp.concatenate([xf[0:r0], xf[1:r0 + 1], xf[2:r0 + 2]], axis=1)
    p0 = _banded_matmul_pool(t0, bd_ref[0:384], bs_ref[0:1], bblk, 32, 17,
                             rows=36)
    t1 = _kh_pieces(p0, 32, 9)
    p1 = _banded_matmul_pool(t1, bd_ref[384:2688], bs_ref[1:2], bblk, 64, 9)
    t2 = _kh_pieces(p1, 64, 5)
    p2 = _banded_matmul_pool(t2, bd_ref[2688:5376], bs_ref[2:3], bblk, 128, 5)
    p2w = jnp.stack([p2[:, :, w * 128:(w + 1) * 128] for w in range(5)], axis=2)
    p3 = _direct_layer(p2w, wx_ref[0:1152], bs_ref[3:4, :256], 256, 3)
    p4 = _direct_layer(p3, wx_ref[1152:3456], bs_ref[4:5, :512], 512, 2)
    ha = None
    hb = None
    for idx, (hh, ww) in enumerate(((0, 0), (0, 1), (1, 0), (1, 1))):
        xp = p4[:, hh, ww, :]
        da = jnp.dot(xp, wx_ref[3456 + 512 * idx:3456 + 512 * (idx + 1)],
                     preferred_element_type=jnp.float32)
        db = jnp.dot(xp, wx_ref[5504 + 512 * idx:5504 + 512 * (idx + 1)],
                     preferred_element_type=jnp.float32)
        ha = da if ha is None else ha + da
        hb = db if hb is None else hb + db
    hcat = jnp.concatenate([ha, hb], axis=1)
    hcat = (hcat + bs_ref[5:6, :1024]).astype(jnp.bfloat16)
    out = jnp.dot(hcat, wx_ref[7552:8576, :128],
                  preferred_element_type=jnp.float32)
    o_ref[...] = out + bs_ref[6:7, :128]


def kernel(x, conv0_w, conv0_b, conv1_w, conv1_b, conv2_w, conv2_b,
           conv3_w, conv3_b, conv4_w, conv4_b, fc1_w, fc1_b, fc2_w, fc2_b):
    b = x.shape[0]
    bblk = _BBLK
    xh = jnp.transpose(x, (0, 2, 3, 1)).astype(jnp.bfloat16)
    xh = jnp.pad(xh, ((0, 0), (2, 2), (2, 2), (0, 0)))
    xh = xh.reshape(b, 36, 108)

    bd = jnp.concatenate(
        [_band(conv0_w, 32), _band(conv1_w, 17), _band(conv2_w, 9)], axis=0)
    w3 = jnp.pad(conv3_w.reshape(9 * 128, 256), ((0, 0), (0, 256)))
    w4 = conv4_w.reshape(9 * 256, 512)
    fw1 = fc1_w.reshape(512, 2, 2, 1024).transpose(1, 2, 0, 3).reshape(2048, 1024)
    fw2 = jnp.pad(fc2_w, ((0, 0), (0, 502)))
    wx = jnp.concatenate([w3, w4, fw1[:, :512], fw1[:, 512:], fw2], axis=0)
    bs = jnp.stack([
        _band_bias(conv0_b, 32),
        _band_bias(conv1_b, 17),
        _band_bias(conv2_b, 9),
        jnp.pad(conv3_b.astype(jnp.float32), (0, 1024)),
        jnp.pad(conv4_b.astype(jnp.float32), (0, 768)),
        jnp.pad(fc1_b.astype(jnp.float32), (0, 256)),
        jnp.pad(fc2_b.astype(jnp.float32), (0, 1270)),
    ], axis=0)

    vmem = pl.BlockSpec(memory_space=pltpu.VMEM)
    out = pl.pallas_call(
        _fused_kernel,
        out_shape=jax.ShapeDtypeStruct((b, 128), jnp.float32),
        grid=(b // bblk,),
        in_specs=[pl.BlockSpec((bblk, 36, 108), lambda i: (i, 0, 0))] + [vmem] * 3,
        out_specs=pl.BlockSpec((bblk, 128), lambda i: (i, 0)),
        compiler_params=pltpu.CompilerParams(
            dimension_semantics=("parallel",),
        ),
        name="fused_cnn",
    )(xh, bd, wx, bs)
    return out[:, :10]


# speedup vs baseline: 1.3310x; 1.1065x over previous
import jax
import jax.numpy as jnp
from jax.experimental import pallas as pl
from jax.experimental.pallas import tpu as pltpu

_BBLK = 32


def _ceil128(n):
    return -(-n // 128) * 128


def _band(w, s_in):
    cin, cout = w.shape[2], w.shape[3]
    wpad = s_in + 4
    sp = (s_in + 2) // 2
    wpc = sp * cout
    hh = _ceil128(wpc)
    wf = w.astype(jnp.float32)
    wv = jnp.arange(wpad)[:, None, None, None]
    tgt = (2 * jnp.arange(sp)[None, None, :, None]
           + jnp.arange(2)[None, :, None, None]
           + jnp.arange(3)[None, None, None, :])
    sel = (wv == tgt).astype(jnp.float32)
    band = jnp.einsum('wqpk,hkio->hwiqpo', sel, wf)
    band = band.reshape(3, wpad * cin, 2, wpc)
    band = jnp.pad(band, ((0, 0), (0, 0), (0, 0), (0, hh - wpc)))
    kp = _ceil128(wpad * cin)
    band = jnp.pad(band, ((0, 0), (0, kp - wpad * cin), (0, 0), (0, 0)))
    return band.reshape(3 * kp, 2 * hh).astype(jnp.bfloat16)


def _band_bias(b, s_in):
    cout = b.shape[0]
    sp = (s_in + 2) // 2
    wpc = sp * cout
    hh = _ceil128(wpc)
    bb = jnp.pad(jnp.tile(b.astype(jnp.float32), sp), (0, hh - wpc))
    return jnp.tile(bb, 2)


def _banded_matmul_pool(t, band, bias, scr, bblk, cout, sp, rows=None):
    rows = 2 * sp if rows is None else rows
    hh = band.shape[1] // 2
    wpc = sp * cout
    acc = jnp.dot(t, band, preferred_element_type=jnp.float32)
    acc = acc + bias
    y = jnp.maximum(acc[:, :hh], acc[:, hh:])[:, :wpc]
    scr[...] = y.reshape(bblk, rows // 2, 2, wpc)
    y = jnp.maximum(scr[:, :sp, 0, :], scr[:, :sp, 1, :])
    return y.astype(jnp.bfloat16)


def _kh_pieces(p, c, s_out):
    bblk, s_in, _ = p.shape
    wpad_c = (s_in + 4) * c
    kp = _ceil128(wpad_c)
    h = 2 * s_out
    pieces = []
    for s in range(3):
        lo, hi = s - 2, s - 2 + h
        top = max(0, -lo)
        seg = p[:, max(0, lo):min(s_in, hi)]
        bot = h - top - (min(s_in, hi) - max(0, lo))
        q = jnp.pad(seg, ((0, 0), (top, bot), (2 * c, kp - wpad_c + 2 * c)))
        pieces.append(q.reshape(bblk * h, kp))
    return jnp.concatenate(pieces, axis=1)


def _direct_layer(p4, w, bias, cout, sp):
    bblk, s_in, _, cin = p4.shape
    h = 2 * sp
    r = bblk * h * h
    t = jnp.pad(p4, ((0, 0), (2, 2), (2, 2), (0, 0)))
    pieces = [t[:, kh:kh + h, kw:kw + h, :].reshape(r, cin)
              for kh in range(3) for kw in range(3)]
    tt = jnp.concatenate(pieces, axis=1)
    acc = jnp.dot(tt, w, preferred_element_type=jnp.float32)[:, :cout]
    acc = acc + bias
    y = acc.reshape(bblk, h, h, cout)
    y = y.reshape(bblk, sp, 2, h, cout).max(axis=2)
    y = y.reshape(bblk, sp, sp, 2, cout).max(axis=3)
    return y.astype(jnp.bfloat16)


def _fused_kernel(x_ref, bd_ref, wx_ref, bs_ref, o_ref, scr0, scr1, scr2):
    bblk = x_ref.shape[0]
    r0 = bblk * 36
    xf = jnp.pad(x_ref[...].reshape(r0, 108), ((0, 2), (0, 20)))
    t0 = jnp.concatenate([xf[0:r0], xf[1:r0 + 1], xf[2:r0 + 2]], axis=1)
    p0 = _banded_matmul_pool(t0, bd_ref[0:384], bs_ref[0:1], scr0, bblk, 32,
                             17, rows=36)
    t1 = _kh_pieces(p0, 32, 9)
    p1 = _banded_matmul_pool(t1, bd_ref[384:2688], bs_ref[1:2], scr1, bblk,
                             64, 9)
    t2 = _kh_pieces(p1, 64, 5)
    p2 = _banded_matmul_pool(t2, bd_ref[2688:5376], bs_ref[2:3], scr2, bblk,
                             128, 5)
    p2w = jnp.stack([p2[:, :, w * 128:(w + 1) * 128] for w in range(5)], axis=2)
    p3 = _direct_layer(p2w, wx_ref[0:1152], bs_ref[3:4, :256], 256, 3)
    p4 = _direct_layer(p3, wx_ref[1152:3456], bs_ref[4:5, :512], 512, 2)
    ha = None
    hb = None
    for idx, (hh, ww) in enumerate(((0, 0), (0, 1), (1, 0), (1, 1))):
        xp = p4[:, hh, ww, :]
        da = jnp.dot(xp, wx_ref[3456 + 512 * idx:3456 + 512 * (idx + 1)],
                     preferred_element_type=jnp.float32)
        db = jnp.dot(xp, wx_ref[5504 + 512 * idx:5504 + 512 * (idx + 1)],
                     preferred_element_type=jnp.float32)
        ha = da if ha is None else ha + da
        hb = db if hb is None else hb + db
    hcat = jnp.concatenate([ha, hb], axis=1)
    hcat = (hcat + bs_ref[5:6, :1024]).astype(jnp.bfloat16)
    out = jnp.dot(hcat, wx_ref[7552:8576, :128],
                  preferred_element_type=jnp.float32)
    o_ref[...] = out + bs_ref[6:7, :128]


def kernel(x, conv0_w, conv0_b, conv1_w, conv1_b, conv2_w, conv2_b,
           conv3_w, conv3_b, conv4_w, conv4_b, fc1_w, fc1_b, fc2_w, fc2_b):
    b = x.shape[0]
    bblk = _BBLK
    xh = jnp.transpose(x, (0, 2, 3, 1)).astype(jnp.bfloat16)
    xh = jnp.pad(xh, ((0, 0), (2, 2), (2, 2), (0, 0)))
    xh = xh.reshape(b, 36, 108)

    bd = jnp.concatenate(
        [_band(conv0_w, 32), _band(conv1_w, 17), _band(conv2_w, 9)], axis=0)
    w3 = jnp.pad(conv3_w.reshape(9 * 128, 256), ((0, 0), (0, 256)))
    w4 = conv4_w.reshape(9 * 256, 512)
    fw1 = fc1_w.reshape(512, 2, 2, 1024).transpose(1, 2, 0, 3).reshape(2048, 1024)
    fw2 = jnp.pad(fc2_w, ((0, 0), (0, 502)))
    wx = jnp.concatenate([w3, w4, fw1[:, :512], fw1[:, 512:], fw2], axis=0)
    bs = jnp.stack([
        _band_bias(conv0_b, 32),
        _band_bias(conv1_b, 17),
        _band_bias(conv2_b, 9),
        jnp.pad(conv3_b.astype(jnp.float32), (0, 1024)),
        jnp.pad(conv4_b.astype(jnp.float32), (0, 768)),
        jnp.pad(fc1_b.astype(jnp.float32), (0, 256)),
        jnp.pad(fc2_b.astype(jnp.float32), (0, 1270)),
    ], axis=0)

    vmem = pl.BlockSpec(memory_space=pltpu.VMEM)
    out = pl.pallas_call(
        _fused_kernel,
        out_shape=jax.ShapeDtypeStruct((b, 128), jnp.float32),
        grid=(b // bblk,),
        in_specs=[pl.BlockSpec((bblk, 36, 108), lambda i: (i, 0, 0))] + [vmem] * 3,
        out_specs=pl.BlockSpec((bblk, 128), lambda i: (i, 0)),
        scratch_shapes=[
            pltpu.VMEM((bblk, 18, 2, 17 * 32), jnp.float32),
            pltpu.VMEM((bblk, 9, 2, 9 * 64), jnp.float32),
            pltpu.VMEM((bblk, 5, 2, 5 * 128), jnp.float32),
        ],
        compiler_params=pltpu.CompilerParams(
            dimension_semantics=("parallel",),
        ),
        name="fused_cnn",
    )(xh, bd, wx, bs)
    return out[:, :10]


# speedup vs baseline: 1.3727x; 1.0314x over previous
import jax
import jax.numpy as jnp
from jax.experimental import pallas as pl
from jax.experimental.pallas import tpu as pltpu

_BBLK = 32


def _ceil128(n):
    return -(-n // 128) * 128


def _band(w, s_in):
    cin, cout = w.shape[2], w.shape[3]
    wpad = s_in + 4
    sp = (s_in + 2) // 2
    wpc = sp * cout
    hh = _ceil128(wpc)
    wf = w.astype(jnp.float32)
    wv = jnp.arange(wpad)[:, None, None, None]
    tgt = (2 * jnp.arange(sp)[None, None, :, None]
           + jnp.arange(2)[None, :, None, None]
           + jnp.arange(3)[None, None, None, :])
    sel = (wv == tgt).astype(jnp.float32)
    band = jnp.einsum('wqpk,hkio->hwiqpo', sel, wf)
    band = band.reshape(3, wpad * cin, 2, wpc)
    band = jnp.pad(band, ((0, 0), (0, 0), (0, 0), (0, hh - wpc)))
    kp = _ceil128(wpad * cin)
    band = jnp.pad(band, ((0, 0), (0, kp - wpad * cin), (0, 0), (0, 0)))
    return band.reshape(3 * kp, 2 * hh).astype(jnp.bfloat16)


def _band_bias(b, s_in):
    cout = b.shape[0]
    sp = (s_in + 2) // 2
    wpc = sp * cout
    hh = _ceil128(wpc)
    bb = jnp.pad(jnp.tile(b.astype(jnp.float32), sp), (0, hh - wpc))
    return jnp.tile(bb, 2)


def _banded_matmul_pool(t, band, bias, scr, bblk, cout, sp, rows=None):
    rows = 2 * sp if rows is None else rows
    hh = band.shape[1] // 2
    wpc = sp * cout
    acc = jnp.dot(t, band, preferred_element_type=jnp.float32)
    acc = acc + bias
    y = jnp.maximum(acc[:, :hh], acc[:, hh:])[:, :wpc]
    scr[...] = y.reshape(bblk, rows // 2, 2, wpc)
    y = jnp.maximum(scr[:, :sp, 0, :], scr[:, :sp, 1, :])
    return y.astype(jnp.bfloat16)


def _kh_pieces(p, c, s_out):
    bblk, s_in, _ = p.shape
    wpad_c = (s_in + 4) * c
    kp = _ceil128(wpad_c)
    h = 2 * s_out
    pieces = []
    for s in range(3):
        lo, hi = s - 2, s - 2 + h
        top = max(0, -lo)
        seg = p[:, max(0, lo):min(s_in, hi)]
        bot = h - top - (min(s_in, hi) - max(0, lo))
        q = jnp.pad(seg, ((0, 0), (top, bot), (2 * c, kp - wpad_c + 2 * c)))
        pieces.append(q.reshape(bblk * h, kp))
    return jnp.concatenate(pieces, axis=1)


def _direct_layer(p4, w, bias, scr, cout, sp):
    bblk, s_in, _, cin = p4.shape
    h = 2 * sp
    r = bblk * h * h
    t = jnp.pad(p4, ((0, 0), (2, 2), (2, 2), (0, 0)))
    pieces = [t[:, kh:kh + h, kw:kw + h, :].reshape(r, cin)
              for kh in range(3) for kw in range(3)]
    tt = jnp.concatenate(pieces, axis=1)
    acc = jnp.dot(tt, w, preferred_element_type=jnp.float32)[:, :cout]
    acc = acc + bias
    y = acc.reshape(bblk, h, h, cout)
    y = y.reshape(bblk, sp, 2, h, cout).max(axis=2)
    scr[...] = y.reshape(bblk * sp * sp, 2, cout)
    y = jnp.maximum(scr[:, 0, :], scr[:, 1, :])
    return y.reshape(bblk, sp, sp, cout).astype(jnp.bfloat16)


def _fused_kernel(x_ref, bd_ref, wx_ref, bs_ref, o_ref, scr0, scr1, scr2,
                  scr3, scr4):
    bblk = x_ref.shape[0]
    r0 = bblk * 36
    xf = jnp.pad(x_ref[...].reshape(r0, 108), ((0, 2), (0, 20)))
    t0 = jnp.concatenate([xf[0:r0], xf[1:r0 + 1], xf[2:r0 + 2]], axis=1)
    p0 = _banded_matmul_pool(t0, bd_ref[0:384], bs_ref[0:1], scr0, bblk, 32,
                             17, rows=36)
    t1 = _kh_pieces(p0, 32, 9)
    p1 = _banded_matmul_pool(t1, bd_ref[384:2688], bs_ref[1:2], scr1, bblk,
                             64, 9)
    t2 = _kh_pieces(p1, 64, 5)
    p2 = _banded_matmul_pool(t2, bd_ref[2688:5376], bs_ref[2:3], scr2, bblk,
                             128, 5)
    p2w = jnp.stack([p2[:, :, w * 128:(w + 1) * 128] for w in range(5)], axis=2)
    p3 = _direct_layer(p2w, wx_ref[0:1152], bs_ref[3:4, :256], scr3, 256, 3)
    p4 = _direct_layer(p3, wx_ref[1152:3456], bs_ref[4:5, :512], scr4, 512, 2)
    ha = None
    hb = None
    for idx, (hh, ww) in enumerate(((0, 0), (0, 1), (1, 0), (1, 1))):
        xp = p4[:, hh, ww, :]
        da = jnp.dot(xp, wx_ref[3456 + 512 * idx:3456 + 512 * (idx + 1)],
                     preferred_element_type=jnp.float32)
        db = jnp.dot(xp, wx_ref[5504 + 512 * idx:5504 + 512 * (idx + 1)],
                     preferred_element_type=jnp.float32)
        ha = da if ha is None else ha + da
        hb = db if hb is None else hb + db
    hcat = jnp.concatenate([ha, hb], axis=1)
    hcat = (hcat + bs_ref[5:6, :1024]).astype(jnp.bfloat16)
    out = jnp.dot(hcat, wx_ref[7552:8576, :128],
                  preferred_element_type=jnp.float32)
    o_ref[...] = out + bs_ref[6:7, :128]


def kernel(x, conv0_w, conv0_b, conv1_w, conv1_b, conv2_w, conv2_b,
           conv3_w, conv3_b, conv4_w, conv4_b, fc1_w, fc1_b, fc2_w, fc2_b):
    b = x.shape[0]
    bblk = _BBLK
    xh = jnp.transpose(x, (0, 2, 3, 1)).astype(jnp.bfloat16)
    xh = jnp.pad(xh, ((0, 0), (2, 2), (2, 2), (0, 0)))
    xh = xh.reshape(b, 36, 108)

    bd = jnp.concatenate(
        [_band(conv0_w, 32), _band(conv1_w, 17), _band(conv2_w, 9)], axis=0)
    w3 = jnp.pad(conv3_w.reshape(9 * 128, 256), ((0, 0), (0, 256)))
    w4 = conv4_w.reshape(9 * 256, 512)
    fw1 = fc1_w.reshape(512, 2, 2, 1024).transpose(1, 2, 0, 3).reshape(2048, 1024)
    fw2 = jnp.pad(fc2_w, ((0, 0), (0, 502)))
    wx = jnp.concatenate([w3, w4, fw1[:, :512], fw1[:, 512:], fw2], axis=0)
    bs = jnp.stack([
        _band_bias(conv0_b, 32),
        _band_bias(conv1_b, 17),
        _band_bias(conv2_b, 9),
        jnp.pad(conv3_b.astype(jnp.float32), (0, 1024)),
        jnp.pad(conv4_b.astype(jnp.float32), (0, 768)),
        jnp.pad(fc1_b.astype(jnp.float32), (0, 256)),
        jnp.pad(fc2_b.astype(jnp.float32), (0, 1270)),
    ], axis=0)

    vmem = pl.BlockSpec(memory_space=pltpu.VMEM)
    out = pl.pallas_call(
        _fused_kernel,
        out_shape=jax.ShapeDtypeStruct((b, 128), jnp.float32),
        grid=(b // bblk,),
        in_specs=[pl.BlockSpec((bblk, 36, 108), lambda i: (i, 0, 0))] + [vmem] * 3,
        out_specs=pl.BlockSpec((bblk, 128), lambda i: (i, 0)),
        scratch_shapes=[
            pltpu.VMEM((bblk, 18, 2, 17 * 32), jnp.float32),
            pltpu.VMEM((bblk, 9, 2, 9 * 64), jnp.float32),
            pltpu.VMEM((bblk, 5, 2, 5 * 128), jnp.float32),
            pltpu.VMEM((bblk * 9, 2, 256), jnp.float32),
            pltpu.VMEM((bblk * 4, 2, 512), jnp.float32),
        ],
        compiler_params=pltpu.CompilerParams(
            dimension_semantics=("parallel",),
        ),
        name="fused_cnn",
    )(xh, bd, wx, bs)
    return out[:, :10]
